# baseline (speedup 1.0000x reference)
import jax
import jax.numpy as jnp
from jax.experimental import pallas as pl

TAU_CONVERGE = 8.0


def _snap_kernel(zp_ref, zc_ref, cb_ref, csq_ref, out_ref):
    zc = zc_ref[0]
    zp = zp_ref[0]
    cb = cb_ref[0]
    c_sq = csq_ref[0]
    diff = zc - zp
    vel = jnp.sqrt(jnp.sum(diff * diff, axis=0, keepdims=True))
    converged = vel < TAU_CONVERGE
    dots = jax.lax.dot_general(cb, zc, (((1,), (0,)), ((), ())),
                               preferred_element_type=jnp.float32)
    dists = c_sq - 2.0 * dots
    mn = jnp.min(dists, axis=0, keepdims=True)
    onehot = (dists == mn).astype(jnp.float32)
    cnt = jnp.sum(onehot, axis=0, keepdims=True)
    entries = jax.lax.dot_general(cb, onehot, (((0,), (0,)), ((), ())),
                                  preferred_element_type=jnp.float32)
    entries = entries / cnt
    out_ref[0] = jnp.where(converged, entries, zc)


@jax.jit
def kernel(z_prev, z_current, codebook):
    B, L, dim = z_current.shape
    H, M, d = codebook.shape
    N = B * L
    TL = 1024
    zp = z_prev.reshape(N, H, d).transpose(1, 2, 0)
    zc = z_current.reshape(N, H, d).transpose(1, 2, 0)
    c_sq = jnp.sum(codebook * codebook, axis=-1, keepdims=True)
    out = pl.pallas_call(
        _snap_kernel,
        grid=(H, N // TL),
        in_specs=[
            pl.BlockSpec((1, d, TL), lambda h, i: (h, 0, i)),
            pl.BlockSpec((1, d, TL), lambda h, i: (h, 0, i)),
            pl.BlockSpec((1, M, d), lambda h, i: (h, 0, 0)),
            pl.BlockSpec((1, M, 1), lambda h, i: (h, 0, 0)),
        ],
        out_specs=pl.BlockSpec((1, d, TL), lambda h, i: (h, 0, i)),
        out_shape=jax.ShapeDtypeStruct((H, d, N), jnp.float32),
    )(zp, zc, codebook, c_sq)
    return out.transpose(2, 0, 1).reshape(B, L, dim)
